# baseline (speedup 1.0000x reference)
import functools

import jax
import jax.numpy as jnp
from jax import lax
from jax.experimental import pallas as pl
from jax.experimental.pallas import tpu as pltpu
from jax.experimental.pallas import tpu_sc as plsc

N = 50000
D = 128
E = 200000
R = 3

NC = 2
NS = 16
EP_T = 12544
E_PAD = EP_T * NS
K_SC = 3
C = 8448
C_ACC = C + 8
N_PAD = C * K_SC * NC
STRIPE = C // NS
EBUF = EP_T + 384
NVEC2 = EP_T // 32


def _mm_body(x_ref, w_ref, y0_ref, y1_ref, y2_ref):
    xb = x_ref[...]
    y0_ref[...] = jnp.dot(xb, w_ref[0], preferred_element_type=jnp.float32)
    y1_ref[...] = jnp.dot(xb, w_ref[1], preferred_element_type=jnp.float32)
    y2_ref[...] = jnp.dot(xb, w_ref[2], preferred_element_type=jnp.float32)


def _relation_transforms(x, W):
    blk = 400
    return pl.pallas_call(
        _mm_body,
        grid=(N // blk,),
        in_specs=[
            pl.BlockSpec((blk, D), lambda i: (i, 0)),
            pl.BlockSpec((R, D, D), lambda i: (0, 0, 0)),
        ],
        out_specs=[pl.BlockSpec((blk, D), lambda i: (i, 0))] * R,
        out_shape=[jax.ShapeDtypeStruct((N, D), jnp.float32)] * R,
    )(x, W)


def _sc_body(y0, y1, y2, s0, d0, s1, d1, s2, d2, z, h,
             srcb, dstb, gidx, rows, accum, semA, semB):
    cid = lax.axis_index("c")
    sid = lax.axis_index("s")
    ebase = sid * EP_T
    r0 = sid * STRIPE
    tvec = C + (lax.iota(jnp.int32, 16) & 7)

    def build_gidx(off):
        for i2 in range(8):
            gidx[pl.ds(i2 * 16, 16)] = dstb[pl.ds(off + i2 * 16, 16)]

    for k in range(K_SC):
        base = (cid * K_SC + k) * C
        pltpu.sync_copy(z, accum.at[pl.ds(r0, STRIPE)])
        plsc.subcore_barrier()
        for yt, st, dt in ((y0, s0, d0), (y1, s1, d1), (y2, s2, d2)):
            pltpu.sync_copy(st.at[pl.ds(ebase, EP_T)], srcb.at[pl.ds(0, EP_T)])
            pltpu.sync_copy(dt.at[pl.ds(ebase, EP_T)], dstb.at[pl.ds(0, EP_T)])

            def scan_body(i, cnt):
                off = i * 32
                dv1 = dstb[pl.ds(off, 16)]
                sv1 = srcb[pl.ds(off, 16)]
                dv2 = dstb[pl.ds(off + 16, 16)]
                sv2 = srcb[pl.ds(off + 16, 16)]
                loc1 = dv1 - base
                ok1 = (loc1 >= 0) & (loc1 < C)
                loc2 = dv2 - base
                ok2 = (loc2 >= 0) & (loc2 < C)
                n1 = jnp.sum(jnp.where(ok1, 1, 0).astype(jnp.int32))
                n2 = jnp.sum(jnp.where(ok2, 1, 0).astype(jnp.int32))
                plsc.store_compressed(srcb.at[pl.ds(cnt, 16)], sv1, mask=ok1)
                plsc.store_compressed(dstb.at[pl.ds(cnt, 16)], loc1, mask=ok1)
                cnt2 = cnt + n1
                plsc.store_compressed(srcb.at[pl.ds(cnt2, 16)], sv2, mask=ok2)
                plsc.store_compressed(dstb.at[pl.ds(cnt2, 16)], loc2, mask=ok2)
                return cnt2 + n2

            cnt = lax.fori_loop(0, NVEC2, scan_body, 0)
            zero16 = jnp.zeros((16,), jnp.int32)
            for t in range(24):
                srcb[pl.ds(cnt + t * 16, 16)] = zero16
                dstb[pl.ds(cnt + t * 16, 16)] = tvec

            nblk = (cnt + 127) // 128
            npair = jnp.maximum((nblk + 1) // 2, 1)

            def gissue(blk):
                off = pl.multiple_of(blk * 128, 128)
                buf = blk % 2
                pltpu.async_copy(
                    yt.at[srcb.at[pl.ds(off, 128)]], rows.at[buf], semA)

            def gdrain(blk):
                pltpu.make_async_copy(
                    yt.at[srcb.at[pl.ds(0, 128)]], rows.at[blk % 2], semA).wait()

            def scat(blk):
                build_gidx(pl.multiple_of(blk * 128, 128))
                pltpu.sync_copy(rows.at[blk % 2], accum.at[gidx], add=True)

            gissue(0)

            def pair_body(j2, _):
                a = j2 * 2
                gdrain(a)
                gissue(a + 1)
                scat(a)
                gdrain(a + 1)
                gissue(a + 2)
                scat(a + 1)
                return 0

            lax.fori_loop(0, npair, pair_body, 0)
            gdrain(0)

        plsc.subcore_barrier()
        pltpu.sync_copy(accum.at[pl.ds(r0, STRIPE)],
                        h.at[pl.ds(base + r0, STRIPE)])


_sc_aggregate = functools.partial(
    pl.kernel,
    out_type=jax.ShapeDtypeStruct((N_PAD, D), jnp.float32),
    mesh=plsc.VectorSubcoreMesh(core_axis_name="c", subcore_axis_name="s"),
    scratch_types=[
        pltpu.VMEM((EBUF,), jnp.int32),
        pltpu.VMEM((EBUF,), jnp.int32),
        pltpu.VMEM((128,), jnp.int32),
        pltpu.VMEM((2, 128, D), jnp.float32),
        pltpu.VMEM_SHARED((C_ACC, D), jnp.float32),
        pltpu.SemaphoreType.DMA,
        pltpu.SemaphoreType.DMA,
    ],
    compiler_params=pltpu.CompilerParams(needs_layout_passes=False),
)(_sc_body)


def _pad_edges(ei):
    pad = E_PAD - E
    src = jnp.concatenate([ei[0], jnp.zeros((pad,), jnp.int32)])
    dst = jnp.concatenate([ei[1], jnp.full((pad,), 2 ** 30, jnp.int32)])
    return src, dst


def kernel(x, edge_index_r0, edge_index_r1, edge_index_r2, W):
    y0, y1, y2 = _relation_transforms(x, W)
    s0, d0 = _pad_edges(edge_index_r0)
    s1, d1 = _pad_edges(edge_index_r1)
    s2, d2 = _pad_edges(edge_index_r2)
    z = jnp.zeros((STRIPE, D), jnp.float32)
    h_pad = _sc_aggregate(y0, y1, y2, s0, d0, s1, d1, s2, d2, z)
    return h_pad[:N]


# speedup vs baseline: 2.8632x; 2.8632x over previous
import functools

import jax
import jax.numpy as jnp
from jax import lax
from jax.experimental import pallas as pl
from jax.experimental.pallas import tpu as pltpu
from jax.experimental.pallas import tpu_sc as plsc

N = 50000
D = 128
E = 200000
R = 3

NC = 2
NS = 16
EP_T = 12544
E_PAD = EP_T * NS
K_SC = 3
C = 8448
C_ACC = C + 8
N_PAD = C * K_SC * NC
STRIPE = C // NS
EBUF = EP_T + 256
NVEC2 = EP_T // 32


def _mm_body(x_ref, w_ref, y0_ref, y1_ref, y2_ref):
    xb = x_ref[...]
    y0_ref[...] = jnp.dot(xb, w_ref[0], preferred_element_type=jnp.float32)
    y1_ref[...] = jnp.dot(xb, w_ref[1], preferred_element_type=jnp.float32)
    y2_ref[...] = jnp.dot(xb, w_ref[2], preferred_element_type=jnp.float32)


def _relation_transforms(x, W):
    blk = 400
    return pl.pallas_call(
        _mm_body,
        grid=(N // blk,),
        in_specs=[
            pl.BlockSpec((blk, D), lambda i: (i, 0)),
            pl.BlockSpec((R, D, D), lambda i: (0, 0, 0)),
        ],
        out_specs=[pl.BlockSpec((blk, D), lambda i: (i, 0))] * R,
        out_shape=[jax.ShapeDtypeStruct((N, D), jnp.float32)] * R,
    )(x, W)


def _sc_body(y0, y1, y2, s0, d0, s1, d1, s2, d2, z, h,
             srcb, dstb, gidx, rows, accum, semA, semB):
    cid = lax.axis_index("c")
    sid = lax.axis_index("s")
    ebase = sid * EP_T
    r0 = sid * STRIPE
    tvec = C + (lax.iota(jnp.int32, 16) & 7)

    def build_gidx(off):
        for i2 in range(8):
            gidx[pl.ds(i2 * 16, 16)] = dstb[pl.ds(off + i2 * 16, 16)]

    for k in range(K_SC):
        base = (cid * K_SC + k) * C
        pltpu.sync_copy(z, accum.at[pl.ds(r0, STRIPE)])
        plsc.subcore_barrier()
        for yt, st, dt in ((y0, s0, d0), (y1, s1, d1), (y2, s2, d2)):
            pltpu.sync_copy(st.at[pl.ds(ebase, EP_T)], srcb.at[pl.ds(0, EP_T)])
            pltpu.sync_copy(dt.at[pl.ds(ebase, EP_T)], dstb.at[pl.ds(0, EP_T)])

            def scan_body(i, cnt):
                off = i * 32
                dv1 = dstb[pl.ds(off, 16)]
                sv1 = srcb[pl.ds(off, 16)]
                dv2 = dstb[pl.ds(off + 16, 16)]
                sv2 = srcb[pl.ds(off + 16, 16)]
                loc1 = dv1 - base
                ok1 = (loc1 >= 0) & (loc1 < C)
                loc2 = dv2 - base
                ok2 = (loc2 >= 0) & (loc2 < C)
                n1 = jnp.sum(jnp.where(ok1, 1, 0).astype(jnp.int32))
                n2 = jnp.sum(jnp.where(ok2, 1, 0).astype(jnp.int32))
                plsc.store_compressed(srcb.at[pl.ds(cnt, 16)], sv1, mask=ok1)
                plsc.store_compressed(dstb.at[pl.ds(cnt, 16)], loc1, mask=ok1)
                cnt2 = cnt + n1
                plsc.store_compressed(srcb.at[pl.ds(cnt2, 16)], sv2, mask=ok2)
                plsc.store_compressed(dstb.at[pl.ds(cnt2, 16)], loc2, mask=ok2)
                return cnt2 + n2

            cnt = lax.fori_loop(0, NVEC2, scan_body, 0)
            zero16 = jnp.zeros((16,), jnp.int32)
            for t in range(16):
                srcb[pl.ds(cnt + t * 16, 16)] = zero16
                dstb[pl.ds(cnt + t * 16, 16)] = tvec

            nblk = (cnt + 127) // 128

            def blk_body(j, _):
                off = pl.multiple_of(j * 128, 128)
                cp = pltpu.async_copy(
                    yt.at[srcb.at[pl.ds(off, 128)]], rows.at[0], semA)
                build_gidx(off)
                cp.wait()
                pltpu.sync_copy(rows.at[0], accum.at[gidx], add=True)
                return 0

            lax.fori_loop(0, nblk, blk_body, 0)
        plsc.subcore_barrier()
        pltpu.sync_copy(accum.at[pl.ds(r0, STRIPE)],
                        h.at[pl.ds(base + r0, STRIPE)])


_sc_aggregate = functools.partial(
    pl.kernel,
    out_type=jax.ShapeDtypeStruct((N_PAD, D), jnp.float32),
    mesh=plsc.VectorSubcoreMesh(core_axis_name="c", subcore_axis_name="s"),
    scratch_types=[
        pltpu.VMEM((EBUF,), jnp.int32),
        pltpu.VMEM((EBUF,), jnp.int32),
        pltpu.VMEM((128,), jnp.int32),
        pltpu.VMEM((2, 128, D), jnp.float32),
        pltpu.VMEM_SHARED((C_ACC, D), jnp.float32),
        pltpu.SemaphoreType.DMA,
        pltpu.SemaphoreType.DMA,
    ],
    compiler_params=pltpu.CompilerParams(needs_layout_passes=False),
)(_sc_body)


def _pad_edges(ei):
    pad = E_PAD - E
    src = jnp.concatenate([ei[0], jnp.zeros((pad,), jnp.int32)])
    dst = jnp.concatenate([ei[1], jnp.full((pad,), 2 ** 30, jnp.int32)])
    return src, dst


def kernel(x, edge_index_r0, edge_index_r1, edge_index_r2, W):
    y0, y1, y2 = _relation_transforms(x, W)
    s0, d0 = _pad_edges(edge_index_r0)
    s1, d1 = _pad_edges(edge_index_r1)
    s2, d2 = _pad_edges(edge_index_r2)
    z = jnp.zeros((STRIPE, D), jnp.float32)
    h_pad = _sc_aggregate(y0, y1, y2, s0, d0, s1, d1, s2, d2, z)
    return h_pad[:N]


# speedup vs baseline: 2.9220x; 1.0205x over previous
import functools

import jax
import jax.numpy as jnp
from jax import lax
from jax.experimental import pallas as pl
from jax.experimental.pallas import tpu as pltpu
from jax.experimental.pallas import tpu_sc as plsc

N = 50000
D = 128
E = 200000
R = 3

NC = 2
NS = 16
EP_T = 12544
E_PAD = EP_T * NS
K_SC = 3
C = 8448
C_ACC = C + 8
N_PAD = C * K_SC * NC
STRIPE = C // NS
EBUF = EP_T + 256
NVEC4 = EP_T // 64


def _mm_body(x_ref, w_ref, y0_ref, y1_ref, y2_ref):
    xb = x_ref[...]
    y0_ref[...] = jnp.dot(xb, w_ref[0], preferred_element_type=jnp.float32)
    y1_ref[...] = jnp.dot(xb, w_ref[1], preferred_element_type=jnp.float32)
    y2_ref[...] = jnp.dot(xb, w_ref[2], preferred_element_type=jnp.float32)


def _relation_transforms(x, W):
    blk = 400
    return pl.pallas_call(
        _mm_body,
        grid=(N // blk,),
        in_specs=[
            pl.BlockSpec((blk, D), lambda i: (i, 0)),
            pl.BlockSpec((R, D, D), lambda i: (0, 0, 0)),
        ],
        out_specs=[pl.BlockSpec((blk, D), lambda i: (i, 0))] * R,
        out_shape=[jax.ShapeDtypeStruct((N, D), jnp.float32)] * R,
    )(x, W)


def _sc_body(y0, y1, y2, s0, d0, s1, d1, s2, d2, z, h,
             srcb, dstb, gidx, rows, accum, semA, semB):
    cid = lax.axis_index("c")
    sid = lax.axis_index("s")
    ebase = sid * EP_T
    r0 = sid * STRIPE
    tvec = C + (lax.iota(jnp.int32, 16) & 7)

    def build_gidx(off):
        for i2 in range(8):
            gidx[pl.ds(i2 * 16, 16)] = dstb[pl.ds(off + i2 * 16, 16)]

    for k in range(K_SC):
        base = (cid * K_SC + k) * C
        pltpu.sync_copy(z, accum.at[pl.ds(r0, STRIPE)])
        plsc.subcore_barrier()
        for yt, st, dt in ((y0, s0, d0), (y1, s1, d1), (y2, s2, d2)):
            pltpu.sync_copy(st.at[pl.ds(ebase, EP_T)], srcb.at[pl.ds(0, EP_T)])
            pltpu.sync_copy(dt.at[pl.ds(ebase, EP_T)], dstb.at[pl.ds(0, EP_T)])

            def scan_body(i, cnt):
                off = i * 64
                dv = [dstb[pl.ds(off + 16 * u, 16)] for u in range(4)]
                sv = [srcb[pl.ds(off + 16 * u, 16)] for u in range(4)]
                loc = [d - base for d in dv]
                ok = [(l >= 0) & (l < C) for l in loc]
                ns = [jnp.sum(jnp.where(o, 1, 0).astype(jnp.int32)) for o in ok]
                for u in range(4):
                    plsc.store_compressed(srcb.at[pl.ds(cnt, 16)], sv[u],
                                          mask=ok[u])
                    plsc.store_compressed(dstb.at[pl.ds(cnt, 16)], loc[u],
                                          mask=ok[u])
                    cnt = cnt + ns[u]
                return cnt

            cnt = lax.fori_loop(0, NVEC4, scan_body, 0)
            zero16 = jnp.zeros((16,), jnp.int32)
            for t in range(16):
                srcb[pl.ds(cnt + t * 16, 16)] = zero16
                dstb[pl.ds(cnt + t * 16, 16)] = tvec

            nblk = (cnt + 127) // 128

            def blk_body(j, _):
                off = pl.multiple_of(j * 128, 128)
                cp = pltpu.async_copy(
                    yt.at[srcb.at[pl.ds(off, 128)]], rows.at[0], semA)
                build_gidx(off)
                cp.wait()
                pltpu.sync_copy(rows.at[0], accum.at[gidx], add=True)
                return 0

            lax.fori_loop(0, nblk, blk_body, 0)
        plsc.subcore_barrier()
        pltpu.sync_copy(accum.at[pl.ds(r0, STRIPE)],
                        h.at[pl.ds(base + r0, STRIPE)])


_sc_aggregate = functools.partial(
    pl.kernel,
    out_type=jax.ShapeDtypeStruct((N_PAD, D), jnp.float32),
    mesh=plsc.VectorSubcoreMesh(core_axis_name="c", subcore_axis_name="s"),
    scratch_types=[
        pltpu.VMEM((EBUF,), jnp.int32),
        pltpu.VMEM((EBUF,), jnp.int32),
        pltpu.VMEM((128,), jnp.int32),
        pltpu.VMEM((2, 128, D), jnp.float32),
        pltpu.VMEM_SHARED((C_ACC, D), jnp.float32),
        pltpu.SemaphoreType.DMA,
        pltpu.SemaphoreType.DMA,
    ],
    compiler_params=pltpu.CompilerParams(needs_layout_passes=False),
)(_sc_body)


def _pad_edges(ei):
    pad = E_PAD - E
    src = jnp.concatenate([ei[0], jnp.zeros((pad,), jnp.int32)])
    dst = jnp.concatenate([ei[1], jnp.full((pad,), 2 ** 30, jnp.int32)])
    return src, dst


def kernel(x, edge_index_r0, edge_index_r1, edge_index_r2, W):
    y0, y1, y2 = _relation_transforms(x, W)
    s0, d0 = _pad_edges(edge_index_r0)
    s1, d1 = _pad_edges(edge_index_r1)
    s2, d2 = _pad_edges(edge_index_r2)
    z = jnp.zeros((STRIPE, D), jnp.float32)
    h_pad = _sc_aggregate(y0, y1, y2, s0, d0, s1, d1, s2, d2, z)
    return h_pad[:N]


# speedup vs baseline: 2.9316x; 1.0033x over previous
import functools

import jax
import jax.numpy as jnp
from jax import lax
from jax.experimental import pallas as pl
from jax.experimental.pallas import tpu as pltpu
from jax.experimental.pallas import tpu_sc as plsc

N = 50000
D = 128
E = 200000
R = 3

NC = 2
NS = 16
EP_T = 12544
E_PAD = EP_T * NS
K_SC = 3
C = 8448
C_ACC = C + 8
N_PAD = C * K_SC * NC
STRIPE = C // NS
EBUF = EP_T + 256
NVEC4 = EP_T // 64


def _mm_body(x_ref, w_ref, y0_ref, y1_ref, y2_ref):
    xb = x_ref[...]
    y0_ref[...] = jnp.dot(xb, w_ref[0], preferred_element_type=jnp.float32)
    y1_ref[...] = jnp.dot(xb, w_ref[1], preferred_element_type=jnp.float32)
    y2_ref[...] = jnp.dot(xb, w_ref[2], preferred_element_type=jnp.float32)


def _relation_transforms(x, W):
    blk = 400
    return pl.pallas_call(
        _mm_body,
        grid=(N // blk,),
        in_specs=[
            pl.BlockSpec((blk, D), lambda i: (i, 0)),
            pl.BlockSpec((R, D, D), lambda i: (0, 0, 0)),
        ],
        out_specs=[pl.BlockSpec((blk, D), lambda i: (i, 0))] * R,
        out_shape=[jax.ShapeDtypeStruct((N, D), jnp.float32)] * R,
    )(x, W)


def _sc_body(y0, y1, y2, s0, d0, s1, d1, s2, d2, z, h,
             srcb, dstb, gidx, rows, accum, semA):
    cid = lax.axis_index("c")
    sid = lax.axis_index("s")
    ebase = sid * EP_T
    r0 = sid * STRIPE
    tvec = C + (lax.iota(jnp.int32, 16) & 7)

    def build_gidx(off):
        for i2 in range(8):
            gidx[pl.ds(i2 * 16, 16)] = dstb[pl.ds(off + i2 * 16, 16)]

    for k in range(K_SC):
        base = (cid * K_SC + k) * C
        pltpu.sync_copy(z, accum.at[pl.ds(r0, STRIPE)])
        plsc.subcore_barrier()
        for yt, st, dt in ((y0, s0, d0), (y1, s1, d1), (y2, s2, d2)):
            pltpu.sync_copy(st.at[pl.ds(ebase, EP_T)], srcb.at[pl.ds(0, EP_T)])
            pltpu.sync_copy(dt.at[pl.ds(ebase, EP_T)], dstb.at[pl.ds(0, EP_T)])

            def scan_body(i, cnt):
                off = i * 64
                dv = [dstb[pl.ds(off + 16 * u, 16)] for u in range(4)]
                sv = [srcb[pl.ds(off + 16 * u, 16)] for u in range(4)]
                loc = [d - base for d in dv]
                ok = [(l >= 0) & (l < C) for l in loc]
                ns = [jnp.sum(jnp.where(o, 1, 0).astype(jnp.int32)) for o in ok]
                for u in range(4):
                    plsc.store_compressed(srcb.at[pl.ds(cnt, 16)], sv[u],
                                          mask=ok[u])
                    plsc.store_compressed(dstb.at[pl.ds(cnt, 16)], loc[u],
                                          mask=ok[u])
                    cnt = cnt + ns[u]
                return cnt

            cnt = lax.fori_loop(0, NVEC4, scan_body, 0)
            zero16 = jnp.zeros((16,), jnp.int32)
            for t in range(16):
                srcb[pl.ds(cnt + t * 16, 16)] = zero16
                dstb[pl.ds(cnt + t * 16, 16)] = tvec

            nblk = (cnt + 127) // 128

            def blk_body(j, _):
                off = pl.multiple_of(j * 128, 128)
                cp = pltpu.async_copy(
                    yt.at[srcb.at[pl.ds(off, 128)]], rows, semA)
                build_gidx(off)
                cp.wait()
                pltpu.sync_copy(rows, accum.at[gidx], add=True)
                return 0

            lax.fori_loop(0, nblk, blk_body, 0)
        plsc.subcore_barrier()
        pltpu.sync_copy(accum.at[pl.ds(r0, STRIPE)],
                        h.at[pl.ds(base + r0, STRIPE)])


_sc_aggregate = functools.partial(
    pl.kernel,
    out_type=jax.ShapeDtypeStruct((N_PAD, D), jnp.float32),
    mesh=plsc.VectorSubcoreMesh(core_axis_name="c", subcore_axis_name="s"),
    scratch_types=[
        pltpu.VMEM((EBUF,), jnp.int32),
        pltpu.VMEM((EBUF,), jnp.int32),
        pltpu.VMEM((128,), jnp.int32),
        pltpu.VMEM((128, D), jnp.float32),
        pltpu.VMEM_SHARED((C_ACC, D), jnp.float32),
        pltpu.SemaphoreType.DMA,
    ],
    compiler_params=pltpu.CompilerParams(needs_layout_passes=False),
)(_sc_body)


def _pad_edges(ei):
    pad = E_PAD - E
    src = jnp.concatenate([ei[0], jnp.zeros((pad,), jnp.int32)])
    dst = jnp.concatenate([ei[1], jnp.full((pad,), 2 ** 30, jnp.int32)])
    return src, dst


def kernel(x, edge_index_r0, edge_index_r1, edge_index_r2, W):
    y0, y1, y2 = _relation_transforms(x, W)
    s0, d0 = _pad_edges(edge_index_r0)
    s1, d1 = _pad_edges(edge_index_r1)
    s2, d2 = _pad_edges(edge_index_r2)
    z = jnp.zeros((STRIPE, D), jnp.float32)
    h_pad = _sc_aggregate(y0, y1, y2, s0, d0, s1, d1, s2, d2, z)
    return h_pad[:N]
